# baseline (speedup 1.0000x reference)
import functools

import numpy as np
import jax
import jax.numpy as jnp
from jax import lax
from jax.experimental import pallas as pl
from jax.experimental.pallas import tpu as pltpu
from jax.experimental.pallas import tpu_sc as plsc

_LANES = 16
_HALF = 112

_PERM_CACHE = {}


def _full_index(B, N):
    key = (B, N)
    if key not in _PERM_CACHE:
        cpu = jax.local_devices(backend="cpu")[0]
        with jax.default_device(cpu), jax.ensure_compile_time_eval():
            r = np.asarray(jax.random.permutation(jax.random.key(42), N))
        idx = (np.arange(B, dtype=np.int64)[:, None] * N + r[None, :]).reshape(-1)
        _PERM_CACHE[key] = idx.astype(np.int32)
    return _PERM_CACHE[key]


def _copy_rows(src_ref, dst_ref, n_rows, width):
    def body(r, carry):
        for c in range(width // _LANES):
            dst_ref[r, pl.ds(c * _LANES, _LANES)] = src_ref[
                r, pl.ds(c * _LANES, _LANES)
            ]
        return carry

    lax.fori_loop(0, n_rows, body, 0)


@functools.lru_cache(maxsize=None)
def _make_repack(B, H, W, C):
    info = plsc.get_sparse_core_info()
    NW = info.num_cores * info.num_subcores
    NC = info.num_cores
    R = B * H * W
    planes_per_w = B * H // NW

    mesh = plsc.VectorSubcoreMesh(core_axis_name="c", subcore_axis_name="s")

    @functools.partial(
        pl.kernel,
        mesh=mesh,
        out_type=jax.ShapeDtypeStruct((R, 128), jnp.float32),
        scratch_types=[
            pltpu.VMEM((W, C), jnp.float32),
            pltpu.VMEM((W, 128), jnp.float32),
        ],
    )
    def repack(x_hbm, xp_hbm, buf96_v, buf128_v):
        wid = lax.axis_index("s") * NC + lax.axis_index("c")
        b = wid // (NW // B)
        h_base = (wid % (NW // B)) * planes_per_w

        def body(p, carry):
            h = h_base + p
            pltpu.sync_copy(x_hbm.at[b, h], buf96_v)
            _copy_rows(buf96_v, buf128_v, W, C)
            pltpu.sync_copy(buf128_v, xp_hbm.at[pl.ds((b * H + h) * W, W)])
            return carry

        lax.fori_loop(0, planes_per_w, body, 0)

    return repack


@functools.lru_cache(maxsize=None)
def _make_gather(B, H, W, C):
    info = plsc.get_sparse_core_info()
    NW = info.num_cores * info.num_subcores
    NC = info.num_cores
    R = B * H * W
    planes_per_w = B * H // NW
    n_half = W * planes_per_w // _HALF

    mesh = plsc.VectorSubcoreMesh(core_axis_name="c", subcore_axis_name="s")

    @functools.partial(
        pl.kernel,
        mesh=mesh,
        out_type=jax.ShapeDtypeStruct((B, H, W, C), jnp.float32),
        scratch_types=[
            pltpu.VMEM((n_half, _HALF), jnp.int32),
            pltpu.VMEM((W, 128), jnp.float32),
            pltpu.VMEM((W, C), jnp.float32),
            pltpu.SemaphoreType.DMA,
        ],
    )
    def gather(xp_hbm, idx_hbm, out_hbm, idx_v, buf128_v, buf96_v, sem):
        wid = lax.axis_index("s") * NC + lax.axis_index("c")
        b = wid // (NW // B)
        h_base = (wid % (NW // B)) * planes_per_w
        pltpu.sync_copy(idx_hbm.at[wid], idx_v)

        def body(p, carry):
            h = h_base + p
            cp0 = pltpu.async_copy(
                xp_hbm.at[idx_v.at[2 * p]], buf128_v.at[pl.ds(0, _HALF)], sem
            )
            cp1 = pltpu.async_copy(
                xp_hbm.at[idx_v.at[2 * p + 1]], buf128_v.at[pl.ds(_HALF, _HALF)], sem
            )
            cp0.wait()
            cp1.wait()
            _copy_rows(buf128_v, buf96_v, W, C)
            pltpu.sync_copy(buf96_v, out_hbm.at[b, h])
            return carry

        lax.fori_loop(0, planes_per_w, body, 0)

    return gather


def kernel(x):
    B, H, W, C = x.shape
    N = H * W
    idx = jnp.asarray(_full_index(B, N).reshape(32, -1, _HALF))
    xp = _make_repack(B, H, W, C)(x)
    return _make_gather(B, H, W, C)(xp, idx)


# speedup vs baseline: 1.0045x; 1.0045x over previous
import functools

import numpy as np
import jax
import jax.numpy as jnp
from jax import lax
from jax.experimental import pallas as pl
from jax.experimental.pallas import tpu as pltpu
from jax.experimental.pallas import tpu_sc as plsc

_LANES = 16
_HALF = 112

_PERM_CACHE = {}


def _full_index(B, N):
    key = (B, N)
    if key not in _PERM_CACHE:
        cpu = jax.local_devices(backend="cpu")[0]
        with jax.default_device(cpu), jax.ensure_compile_time_eval():
            r = np.asarray(jax.random.permutation(jax.random.key(42), N))
        idx = (np.arange(B, dtype=np.int64)[:, None] * N + r[None, :]).reshape(-1)
        _PERM_CACHE[key] = idx.astype(np.int32)
    return _PERM_CACHE[key]


def _copy_rows(src_ref, dst_ref, n_rows, width):
    def body(r, carry):
        for c in range(width // _LANES):
            dst_ref[r, pl.ds(c * _LANES, _LANES)] = src_ref[
                r, pl.ds(c * _LANES, _LANES)
            ]
        return carry

    lax.fori_loop(0, n_rows, body, 0)


@functools.lru_cache(maxsize=None)
def _make_shuffle(B, H, W, C):
    info = plsc.get_sparse_core_info()
    NW = info.num_cores * info.num_subcores
    R = B * H * W
    slabs = NW // B
    planes_per_w = H // slabs
    n_half = W * planes_per_w // _HALF

    mesh = plsc.VectorSubcoreMesh(core_axis_name="c", subcore_axis_name="s")

    @functools.partial(
        pl.kernel,
        mesh=mesh,
        out_type=(
            jax.ShapeDtypeStruct((B, H, W, C), jnp.float32),
            jax.ShapeDtypeStruct((R, 128), jnp.float32),
        ),
        scratch_types=[
            pltpu.VMEM((n_half, _HALF), jnp.int32),
            pltpu.VMEM((W, C), jnp.float32),
            pltpu.VMEM((W, 128), jnp.float32),
            pltpu.SemaphoreType.DMA,
        ],
    )
    def shuffle(x_hbm, idx_hbm, out_hbm, xp_hbm, idx_v, buf96_v, buf128_v, sem):
        cid = lax.axis_index("c")
        sid = lax.axis_index("s")
        b = cid * (B // 2) + sid // slabs
        h_base = (sid % slabs) * planes_per_w
        kidx = b * slabs + sid % slabs
        pltpu.sync_copy(idx_hbm.at[kidx], idx_v)

        def repack(p, carry):
            h = h_base + p
            pltpu.sync_copy(x_hbm.at[b, h], buf96_v)
            _copy_rows(buf96_v, buf128_v, W, C)
            pltpu.sync_copy(buf128_v, xp_hbm.at[pl.ds((b * H + h) * W, W)])
            return carry

        lax.fori_loop(0, planes_per_w, repack, 0)
        plsc.subcore_barrier()

        def gather(p, carry):
            h = h_base + p
            cp0 = pltpu.async_copy(
                xp_hbm.at[idx_v.at[2 * p]], buf128_v.at[pl.ds(0, _HALF)], sem
            )
            cp1 = pltpu.async_copy(
                xp_hbm.at[idx_v.at[2 * p + 1]], buf128_v.at[pl.ds(_HALF, _HALF)], sem
            )
            cp0.wait()
            cp1.wait()
            _copy_rows(buf128_v, buf96_v, W, C)
            pltpu.sync_copy(buf96_v, out_hbm.at[b, h])
            return carry

        lax.fori_loop(0, planes_per_w, gather, 0)

    return shuffle


def kernel(x):
    B, H, W, C = x.shape
    N = H * W
    idx = jnp.asarray(_full_index(B, N).reshape(32, -1, _HALF))
    out, _ = _make_shuffle(B, H, W, C)(x, idx)
    return out
